# Initial kernel scaffold; baseline (speedup 1.0000x reference)
import jax
import jax.numpy as jnp
from jax.experimental import pallas as pl


def kernel(direct, table):
    raise NotImplementedError("write your pallas kernel here")


# speedup vs baseline: 677.4884x; 677.4884x over previous
import functools

import jax
import jax.numpy as jnp
from jax import lax
from jax.experimental import pallas as pl
from jax.experimental.pallas import tpu as pltpu
from jax.experimental.pallas import tpu_sc as plsc

TABLE_N = 256
COST_RATIO = 1000.0
BETA = 0.2

NC, NS, L = 2, 16, 16
NW = NC * NS

N_TOTAL = 2 * 8192 * 2048
PER_W = N_TOTAL // NW
CHUNK = 16384
NCHUNK = PER_W // CHUNK
VECS = CHUNK // L


def _loss_body(direct_hbm, table_hbm, out_hbm, table_v, buf_v, out_v):
    wid = lax.axis_index("s") * NC + lax.axis_index("c")
    pltpu.sync_copy(table_hbm, table_v)
    base = wid * PER_W

    def chunk_body(k, acc):
        pltpu.sync_copy(direct_hbm.at[pl.ds(base + k * CHUNK, CHUNK)], buf_v)

        def vec_body(i, a):
            d = buf_v[pl.ds(i * L, L)]
            idx = jnp.clip(d * 255.0, 0.0, 255.0).astype(jnp.int32)
            t = plsc.load_gather(table_v, [idx])
            diff = d - t
            r = d * 50.0
            neg = jnp.where(d > -BETA, r * r, -1000.0 * d - 100.0)
            p = jnp.where(d >= 0.0, d, neg)
            return a + (diff * diff + p)

        return lax.fori_loop(0, VECS, vec_body, acc)

    acc = lax.fori_loop(0, NCHUNK, chunk_body, jnp.zeros((L,), jnp.float32))
    out_v[...] = acc
    pltpu.sync_copy(out_v, out_hbm.at[wid])


@jax.jit
def kernel(direct, table):
    partials = pl.kernel(
        _loss_body,
        out_type=jax.ShapeDtypeStruct((NW, L), jnp.float32),
        mesh=plsc.VectorSubcoreMesh(
            core_axis_name="c", subcore_axis_name="s",
            num_cores=NC, num_subcores=NS),
        scratch_types=[
            pltpu.VMEM((TABLE_N,), jnp.float32),
            pltpu.VMEM((CHUNK,), jnp.float32),
            pltpu.VMEM((L,), jnp.float32),
        ],
        compiler_params=pltpu.CompilerParams(needs_layout_passes=False),
    )(direct.reshape(-1), table)
    return jnp.sum(partials) / N_TOTAL


# speedup vs baseline: 863.5887x; 1.2747x over previous
import functools

import jax
import jax.numpy as jnp
from jax import lax
from jax.experimental import pallas as pl
from jax.experimental.pallas import tpu as pltpu
from jax.experimental.pallas import tpu_sc as plsc

TABLE_N = 256
COST_RATIO = 1000.0
BETA = 0.2

NC, NS, L = 2, 16, 16
NW = NC * NS

N_TOTAL = 2 * 8192 * 2048
PER_W = N_TOTAL // NW
CHUNK = 16384
NCHUNK = PER_W // CHUNK
VECS = CHUNK // L


def _loss_body(direct_hbm, table_hbm, out_hbm, table_v, buf0, buf1, out_v,
               sem0, sem1):
    wid = lax.axis_index("s") * NC + lax.axis_index("c")
    pltpu.sync_copy(table_hbm, table_v)
    base = wid * PER_W

    def src(k):
        kc = jnp.minimum(k, NCHUNK - 1)
        return direct_hbm.at[pl.ds(base + kc * CHUNK, CHUNK)]

    def compute(buf, a):
        def vec_body(i, acc):
            d = buf[pl.ds(i * L, L)]
            idx = jnp.clip(d * 255.0, 0.0, 255.0).astype(jnp.int32)
            t = plsc.load_gather(table_v, [idx])
            diff = d - t
            r = d * 50.0
            neg = jnp.where(d > -BETA, r * r, -1000.0 * d - 100.0)
            p = jnp.where(d >= 0.0, d, neg)
            return acc + (diff * diff + p)

        return plsc.parallel_loop(0, VECS, 1, unroll=8, carry=a)(vec_body)

    pltpu.async_copy(src(0), buf0, sem0)
    pltpu.async_copy(src(1), buf1, sem1)

    def outer(k2, acc):
        k = 2 * k2
        pltpu.make_async_copy(src(k), buf0, sem0).wait()
        acc = compute(buf0, acc)
        pltpu.async_copy(src(k + 2), buf0, sem0)
        pltpu.make_async_copy(src(k + 1), buf1, sem1).wait()
        acc = compute(buf1, acc)
        pltpu.async_copy(src(k + 3), buf1, sem1)
        return acc

    acc = lax.fori_loop(0, NCHUNK // 2, outer, jnp.zeros((L,), jnp.float32))
    pltpu.make_async_copy(src(NCHUNK), buf0, sem0).wait()
    pltpu.make_async_copy(src(NCHUNK + 1), buf1, sem1).wait()
    out_v[...] = acc
    pltpu.sync_copy(out_v, out_hbm.at[wid])


@jax.jit
def kernel(direct, table):
    partials = pl.kernel(
        _loss_body,
        out_type=jax.ShapeDtypeStruct((NW, L), jnp.float32),
        mesh=plsc.VectorSubcoreMesh(
            core_axis_name="c", subcore_axis_name="s",
            num_cores=NC, num_subcores=NS),
        scratch_types=[
            pltpu.VMEM((TABLE_N,), jnp.float32),
            pltpu.VMEM((CHUNK,), jnp.float32),
            pltpu.VMEM((CHUNK,), jnp.float32),
            pltpu.VMEM((L,), jnp.float32),
            pltpu.SemaphoreType.DMA,
            pltpu.SemaphoreType.DMA,
        ],
        compiler_params=pltpu.CompilerParams(needs_layout_passes=False),
    )(direct.reshape(-1), table)
    return jnp.sum(partials) / N_TOTAL


# speedup vs baseline: 1233.2151x; 1.4280x over previous
import functools

import jax
import jax.numpy as jnp
from jax import lax
from jax.experimental import pallas as pl
from jax.experimental.pallas import tpu as pltpu
from jax.experimental.pallas import tpu_sc as plsc

TABLE_N = 256
BETA = 0.2

NC, NS, L = 2, 16, 16
NW = NC * NS

N_TOTAL = 2 * 8192 * 2048
PER_W = N_TOTAL // NW
CHUNK = 16384
NCHUNK = PER_W // CHUNK
VECS = CHUNK // L

TC_ROWS = 16384
TC_COLS = 2048
TC_BM = 512


def _sc_lookup_body(direct_hbm, table_hbm, out_hbm, table_v, buf0, buf1,
                    out_v, sem0, sem1):
    wid = lax.axis_index("s") * NC + lax.axis_index("c")
    pltpu.sync_copy(table_hbm, table_v)
    base = wid * PER_W

    def src(k):
        kc = jnp.minimum(k, NCHUNK - 1)
        return direct_hbm.at[pl.ds(base + kc * CHUNK, CHUNK)]

    def compute(buf, a):
        def vec_body(i, acc):
            d = buf[pl.ds(i * L, L)]
            idx = jnp.clip(d * 255.0, 0.0, 255.0).astype(jnp.int32)
            t = plsc.load_gather(table_v, [idx])
            diff = d - t
            return acc + diff * diff

        return plsc.parallel_loop(0, VECS, 1, unroll=8, carry=a)(vec_body)

    pltpu.async_copy(src(0), buf0, sem0)
    pltpu.async_copy(src(1), buf1, sem1)

    def outer(k2, acc):
        k = 2 * k2
        pltpu.make_async_copy(src(k), buf0, sem0).wait()
        acc = compute(buf0, acc)
        pltpu.async_copy(src(k + 2), buf0, sem0)
        pltpu.make_async_copy(src(k + 1), buf1, sem1).wait()
        acc = compute(buf1, acc)
        pltpu.async_copy(src(k + 3), buf1, sem1)
        return acc

    acc = lax.fori_loop(0, NCHUNK // 2, outer, jnp.zeros((L,), jnp.float32))
    pltpu.make_async_copy(src(NCHUNK), buf0, sem0).wait()
    pltpu.make_async_copy(src(NCHUNK + 1), buf1, sem1).wait()
    out_v[...] = acc
    pltpu.sync_copy(out_v, out_hbm.at[wid])


def _tc_dense_body(d_ref, out_ref, acc_ref):
    d = d_ref[...]
    r = d * 50.0
    neg = jnp.where(d > -BETA, r * r, -1000.0 * d - 100.0)
    p = jnp.where(d >= 0.0, d, neg)
    s = jnp.sum(p)

    @pl.when(pl.program_id(0) == 0)
    def _init():
        acc_ref[0] = 0.0

    acc_ref[0] += s

    @pl.when(pl.program_id(0) == pl.num_programs(0) - 1)
    def _fin():
        out_ref[0] = acc_ref[0]


@jax.jit
def kernel(direct, table):
    sc_partials = pl.kernel(
        _sc_lookup_body,
        out_type=jax.ShapeDtypeStruct((NW, L), jnp.float32),
        mesh=plsc.VectorSubcoreMesh(
            core_axis_name="c", subcore_axis_name="s",
            num_cores=NC, num_subcores=NS),
        scratch_types=[
            pltpu.VMEM((TABLE_N,), jnp.float32),
            pltpu.VMEM((CHUNK,), jnp.float32),
            pltpu.VMEM((CHUNK,), jnp.float32),
            pltpu.VMEM((L,), jnp.float32),
            pltpu.SemaphoreType.DMA,
            pltpu.SemaphoreType.DMA,
        ],
        compiler_params=pltpu.CompilerParams(needs_layout_passes=False),
    )(direct.reshape(-1), table)

    tc_sum = pl.pallas_call(
        _tc_dense_body,
        grid=(TC_ROWS // TC_BM,),
        in_specs=[pl.BlockSpec((TC_BM, TC_COLS), lambda i: (i, 0))],
        out_specs=pl.BlockSpec(memory_space=pltpu.SMEM),
        out_shape=jax.ShapeDtypeStruct((1,), jnp.float32),
        scratch_shapes=[pltpu.SMEM((1,), jnp.float32)],
    )(direct.reshape(TC_ROWS, TC_COLS))

    return (jnp.sum(sc_partials) + tc_sum[0]) / N_TOTAL


# speedup vs baseline: 1803.5969x; 1.4625x over previous
import functools

import jax
import jax.numpy as jnp
from jax import lax
from jax.experimental import pallas as pl
from jax.experimental.pallas import tpu as pltpu
from jax.experimental.pallas import tpu_sc as plsc

TABLE_N = 256
BETA = 0.2

NC, NS, L = 2, 16, 16
NW = NC * NS

N_TOTAL = 2 * 8192 * 2048
PER_W = N_TOTAL // NW
CHUNK = 16384
NCHUNK = PER_W // CHUNK
VECS = CHUNK // L

TC_ROWS = 16384
TC_COLS = 2048
TC_BM = 512


CH_ROWS = CHUNK // TC_COLS
ROWS_PER_W = PER_W // TC_COLS
VECS_PER_ROW = TC_COLS // L


def _sc_lookup_body(direct_hbm, table_hbm, out_hbm, table_v, buf0, buf1,
                    out_v, sem0, sem1):
    wid = lax.axis_index("s") * NC + lax.axis_index("c")
    pltpu.sync_copy(table_hbm, table_v)
    b = wid // NS
    row0 = (wid % NS) * ROWS_PER_W

    def src(k):
        kc = jnp.minimum(k, NCHUNK - 1)
        return direct_hbm.at[b, pl.ds(row0 + kc * CH_ROWS, CH_ROWS), :]

    def compute(buf, a):
        for r in range(CH_ROWS):
            def vec_body(i, acc):
                d = buf[r, pl.ds(i * L, L)]
                idx = jnp.clip(d * 255.0, 0.0, 255.0).astype(jnp.int32)
                t = plsc.load_gather(table_v, [idx])
                diff = d - t
                return acc + diff * diff

            a = plsc.parallel_loop(0, VECS_PER_ROW, 1, unroll=8,
                                   carry=a)(vec_body)
        return a

    pltpu.async_copy(src(0), buf0, sem0)
    pltpu.async_copy(src(1), buf1, sem1)

    def outer(k2, acc):
        k = 2 * k2
        pltpu.make_async_copy(src(k), buf0, sem0).wait()
        acc = compute(buf0, acc)
        pltpu.async_copy(src(k + 2), buf0, sem0)
        pltpu.make_async_copy(src(k + 1), buf1, sem1).wait()
        acc = compute(buf1, acc)
        pltpu.async_copy(src(k + 3), buf1, sem1)
        return acc

    acc = lax.fori_loop(0, NCHUNK // 2, outer, jnp.zeros((L,), jnp.float32))
    pltpu.make_async_copy(src(NCHUNK), buf0, sem0).wait()
    pltpu.make_async_copy(src(NCHUNK + 1), buf1, sem1).wait()
    out_v[...] = acc
    pltpu.sync_copy(out_v, out_hbm.at[wid])


def _tc_dense_body(d_ref, out_ref, acc_ref):
    d = d_ref[...]
    r = d * 50.0
    neg = jnp.where(d > -BETA, r * r, -1000.0 * d - 100.0)
    p = jnp.where(d >= 0.0, d, neg)
    s = jnp.sum(p)

    @pl.when(pl.program_id(0) == 0)
    def _init():
        acc_ref[0] = 0.0

    acc_ref[0] += s

    @pl.when(pl.program_id(0) == pl.num_programs(0) - 1)
    def _fin():
        out_ref[0] = acc_ref[0]


@jax.jit
def kernel(direct, table):
    sc_partials = pl.kernel(
        _sc_lookup_body,
        out_type=jax.ShapeDtypeStruct((NW, L), jnp.float32),
        mesh=plsc.VectorSubcoreMesh(
            core_axis_name="c", subcore_axis_name="s",
            num_cores=NC, num_subcores=NS),
        scratch_types=[
            pltpu.VMEM((TABLE_N,), jnp.float32),
            pltpu.VMEM((CH_ROWS, TC_COLS), jnp.float32),
            pltpu.VMEM((CH_ROWS, TC_COLS), jnp.float32),
            pltpu.VMEM((L,), jnp.float32),
            pltpu.SemaphoreType.DMA,
            pltpu.SemaphoreType.DMA,
        ],
        compiler_params=pltpu.CompilerParams(needs_layout_passes=False),
    )(direct, table)

    tc_sum = pl.pallas_call(
        _tc_dense_body,
        grid=(TC_ROWS // TC_BM,),
        in_specs=[pl.BlockSpec((TC_BM, TC_COLS), lambda i: (i, 0))],
        out_specs=pl.BlockSpec(memory_space=pltpu.SMEM),
        out_shape=jax.ShapeDtypeStruct((1,), jnp.float32),
        scratch_shapes=[pltpu.SMEM((1,), jnp.float32)],
    )(direct.reshape(TC_ROWS, TC_COLS))

    return (jnp.sum(sc_partials) + tc_sum[0]) / N_TOTAL
